# baseline (speedup 1.0000x reference)
import functools

import jax
import jax.numpy as jnp
from jax import lax
from jax.experimental import pallas as pl
from jax.experimental.pallas import tpu as pltpu
from jax.experimental.pallas import tpu_sc as plsc

_NC, _NS, _L = 2, 16, 16
_NW = _NC * _NS
_R = 4
_NG = 3


def kernel(tokens, emb):
    B, N, C = tokens.shape
    rows_w = N // _NW
    n_groups = rows_w // _R

    mesh = plsc.VectorSubcoreMesh(
        core_axis_name="c", subcore_axis_name="s",
        num_cores=_NC, num_subcores=_NS)

    @functools.partial(
        pl.kernel,
        out_type=jax.ShapeDtypeStruct((B, N, C), jnp.float32),
        mesh=mesh,
        scratch_types=(
            [pltpu.VMEM((_R, C), jnp.float32) for _ in range(_NG * B)]
            + [pltpu.VMEM((_R, C), jnp.float32) for _ in range(2)]
            + [pltpu.SemaphoreType.DMA for _ in range(2 * _NG + 2)]
        ),
    )
    def sc_add(tok_hbm, emb_hbm, out_hbm, *refs):
        tg = [list(refs[k * B:(k + 1) * B]) for k in range(_NG)]
        ev = list(refs[_NG * B:_NG * B + 2])
        sti = list(refs[_NG * B + 2:_NG * B + 2 + _NG])
        sto = list(refs[_NG * B + 2 + _NG:_NG * B + 2 + 2 * _NG])
        se = list(refs[_NG * B + 2 + 2 * _NG:])
        wid = lax.axis_index("s") * _NC + lax.axis_index("c")
        base = wid * rows_w

        def rows(g):
            return pl.ds(base + g * _R, _R)

        def ins(g):
            k = g % _NG
            return [pltpu.async_copy(tok_hbm.at[b, rows(g), :], tg[k][b], sti[k])
                    for b in range(B)]

        def outs(g):
            k = g % _NG
            return [pltpu.async_copy(tg[k][b], out_hbm.at[b, rows(g), :], sto[k])
                    for b in range(B)]

        def emb_in(g):
            return pltpu.async_copy(emb_hbm.at[rows(g), :], ev[g & 1], se[g & 1])

        in_dma = [None] * _NG
        out_dma = [None] * _NG
        emb_dma = [None, None]
        emb_dma[0] = emb_in(0)
        emb_dma[1] = emb_in(1)
        in_dma[0] = ins(0)
        in_dma[1] = ins(1)

        for g in range(n_groups):
            k = g % _NG
            q = g & 1
            if g >= 1 and g + 1 < n_groups:
                emb_dma[(g + 1) & 1] = emb_in(g + 1)
            for d in in_dma[k]:
                d.wait()
            emb_dma[q].wait()

            tgb, evq = tg[k], ev[q]

            @plsc.parallel_loop(0, C // _L, unroll=2)
            def _(j):
                s = pl.ds(j * _L, _L)
                for r in range(_R):
                    e = evq[r, s]
                    for b in range(B):
                        plsc.addupdate(tgb[b].at[r, s], e)

            out_dma[k] = outs(g)

            if g + 2 < n_groups:
                kp = (g + 2) % _NG
                if out_dma[kp] is not None:
                    for d in out_dma[kp]:
                        d.wait()
                in_dma[kp] = ins(g + 2)

        for gl in range(max(0, n_groups - _NG), n_groups):
            for d in out_dma[gl % _NG]:
                d.wait()

    return sc_add(tokens, emb)


# speedup vs baseline: 1.0056x; 1.0056x over previous
import functools

import jax
import jax.numpy as jnp
from jax import lax
from jax.experimental import pallas as pl
from jax.experimental.pallas import tpu as pltpu
from jax.experimental.pallas import tpu_sc as plsc

_NC, _NS, _L = 2, 16, 16
_NW = _NC * _NS
_R = 4
_NG = 3


def kernel(tokens, emb):
    B, N, C = tokens.shape
    rows_w = N // _NW
    n_groups = rows_w // _R

    mesh = plsc.VectorSubcoreMesh(
        core_axis_name="c", subcore_axis_name="s",
        num_cores=_NC, num_subcores=_NS)

    @functools.partial(
        pl.kernel,
        out_type=jax.ShapeDtypeStruct((B, N, C), jnp.float32),
        mesh=mesh,
        scratch_types=(
            [pltpu.VMEM((_R, C), jnp.float32) for _ in range(_NG * B)]
            + [pltpu.VMEM((_R, C), jnp.float32) for _ in range(2)]
            + [pltpu.SemaphoreType.DMA for _ in range(2 * _NG + 2)]
        ),
    )
    def sc_add(tok_hbm, emb_hbm, out_hbm, *refs):
        tg = [list(refs[k * B:(k + 1) * B]) for k in range(_NG)]
        ev = list(refs[_NG * B:_NG * B + 2])
        sti = list(refs[_NG * B + 2:_NG * B + 2 + _NG])
        sto = list(refs[_NG * B + 2 + _NG:_NG * B + 2 + 2 * _NG])
        se = list(refs[_NG * B + 2 + 2 * _NG:])
        wid = lax.axis_index("s") * _NC + lax.axis_index("c")
        base = wid * rows_w

        def rows(g):
            return pl.ds(base + g * _R, _R)

        def ins(g):
            k = g % _NG
            return [pltpu.async_copy(tok_hbm.at[b, rows(g), :], tg[k][b], sti[k])
                    for b in range(B)]

        def outs(g):
            k = g % _NG
            return [pltpu.async_copy(tg[k][b], out_hbm.at[b, rows(g), :], sto[k])
                    for b in range(B)]

        def emb_in(g):
            return pltpu.async_copy(emb_hbm.at[rows(g), :], ev[g & 1], se[g & 1])

        in_dma = [None] * _NG
        out_dma = [None] * _NG
        emb_dma = [None, None]
        emb_dma[0] = emb_in(0)
        emb_dma[1] = emb_in(1)
        in_dma[0] = ins(0)
        in_dma[1] = ins(1)

        for g in range(n_groups):
            k = g % _NG
            q = g & 1
            if g >= 1 and g + 1 < n_groups:
                emb_dma[(g + 1) & 1] = emb_in(g + 1)
            for d in in_dma[k]:
                d.wait()
            emb_dma[q].wait()

            tgb, evq = tg[k], ev[q]

            @plsc.parallel_loop(0, C // _L, unroll=1)
            def _(j):
                s = pl.ds(j * _L, _L)
                for r in range(_R):
                    e = evq[r, s]
                    for b in range(B):
                        plsc.addupdate(tgb[b].at[r, s], e)

            out_dma[k] = outs(g)

            if g + 2 < n_groups:
                kp = (g + 2) % _NG
                if out_dma[kp] is not None:
                    for d in out_dma[kp]:
                        d.wait()
                in_dma[kp] = ins(g + 2)

        for gl in range(max(0, n_groups - _NG), n_groups):
            for d in out_dma[gl % _NG]:
                d.wait()

    return sc_add(tokens, emb)
